# baseline (speedup 1.0000x reference)
import functools

import jax
import jax.numpy as jnp
from jax import lax
from jax.experimental import pallas as pl
from jax.experimental.pallas import tpu as pltpu
from jax.experimental.pallas import tpu_sc as plsc

N_USERS = 1000000
D = 32
B = 16384
NC = 2
NS = 16
NW = NC * NS
BPW = B // NW
L = 16
NWAVE = BPW // L


def _perm(x, idx):
    dnums = lax.GatherDimensionNumbers(
        offset_dims=(), collapsed_slice_dims=(0,), start_index_map=(0,))
    return lax.gather(x, idx[:, None], dnums, slice_sizes=(1,),
                      mode=lax.GatherScatterMode.PROMISE_IN_BOUNDS)


@jax.jit
def _gmf_sc(users, items, embed_user, W):
    mesh = plsc.VectorSubcoreMesh(core_axis_name="c", subcore_axis_name="s")

    @functools.partial(
        pl.kernel,
        mesh=mesh,
        out_type=jax.ShapeDtypeStruct((B,), jnp.float32),
        compiler_params=pltpu.CompilerParams(use_tc_tiling_on_sc=True),
        scratch_types=[
            pltpu.VMEM((BPW,), jnp.int32),
            pltpu.VMEM((BPW,), jnp.int32),
            pltpu.VMEM((2, L, 8, L), jnp.int32),
            pltpu.VMEM((2, L, 8, L), jnp.int32),
            pltpu.VMEM((2, L), jnp.float32),
            pltpu.VMEM((BPW,), jnp.float32),
            pltpu.SemaphoreType.DMA,
            pltpu.SemaphoreType.DMA,
        ],
    )
    def run(users_hbm, items_hbm, table_hbm, w_hbm, out_hbm,
            uidx_v, iidx_v, ublk_v, iblk_v, w_v, out_v, sem0, sem1):
        wid = lax.axis_index("s") * NC + lax.axis_index("c")
        base = wid * BPW
        pltpu.sync_copy(users_hbm.at[pl.ds(base, BPW)], uidx_v)
        pltpu.sync_copy(items_hbm.at[pl.ds(base, BPW)], iidx_v)
        pltpu.sync_copy(w_hbm, w_v)

        def enqueue_wave(w, p, sem):
            uvec = uidx_v[pl.ds(w * L, L)]
            ivec = iidx_v[pl.ds(w * L, L)]
            for l in range(L):
                vu = uvec[l]
                offu = pl.multiple_of((vu >> 3) << 3, 8)
                pltpu.async_copy(
                    table_hbm.at[pl.ds(offu, 8), :], ublk_v.at[p, l], sem)
                vi = ivec[l]
                offi = pl.multiple_of((vi >> 3) << 3, 8)
                pltpu.async_copy(
                    table_hbm.at[pl.ds(offi, 8), :], iblk_v.at[p, l], sem)

        def drain_wave(p, sem):
            for l in range(L):
                pltpu.make_async_copy(
                    table_hbm.at[pl.ds(0, 8), :], ublk_v.at[p, l], sem).wait()
                pltpu.make_async_copy(
                    table_hbm.at[pl.ds(0, 8), :], iblk_v.at[p, l], sem).wait()

        we_v = w_v[0, :]
        wo_v = w_v[1, :]
        himask = jnp.full((L,), -65536, jnp.int32)
        lanes = lax.iota(jnp.int32, L)
        rot8 = lanes ^ 8
        rot4 = lanes ^ 4
        rot2 = lanes ^ 2
        rot1 = lanes ^ 1

        enqueue_wave(0, 0, sem0)

        def pair_step(h, _):
            we = h * 2
            wo = we + 1

            enqueue_wave(wo, 1, sem1)
            drain_wave(0, sem0)
            compute_wave(we, 0)

            @pl.when(h < NWAVE // 2 - 1)
            def _next_even():
                enqueue_wave(wo + 1, 0, sem0)

            drain_wave(1, sem1)
            compute_wave(wo, 1)
            return 0

        def compute_wave(w, p):
            uvec = uidx_v[pl.ds(w * L, L)]
            ivec = iidx_v[pl.ds(w * L, L)]
            acc = jnp.zeros((L,), jnp.float32)
            for l in range(L):
                ru = uvec[l] & 7
                ri = ivec[l] & 7
                uw = ublk_v[p, l, ru, :]
                iw = iblk_v[p, l, ri, :]
                u_e = lax.bitcast_convert_type(uw << 16, jnp.float32)
                u_o = lax.bitcast_convert_type(uw & himask, jnp.float32)
                i_e = lax.bitcast_convert_type(iw << 16, jnp.float32)
                i_o = lax.bitcast_convert_type(iw & himask, jnp.float32)
                s = u_e * i_e * we_v + u_o * i_o * wo_v
                s = s + _perm(s, rot8)
                s = s + _perm(s, rot4)
                s = s + _perm(s, rot2)
                s = s + _perm(s, rot1)
                acc = jnp.where(lanes == l, s, acc)
            out_v[pl.ds(w * L, L)] = acc

        lax.fori_loop(0, NWAVE // 2, pair_step, 0)

        pltpu.sync_copy(out_v, out_hbm.at[pl.ds(base, BPW)])

    t_i32 = lax.bitcast_convert_type(
        embed_user.astype(jnp.bfloat16).reshape(N_USERS, D // 2, 2),
        jnp.int32)
    wsplit = jnp.stack([W[0, 0::2], W[0, 1::2]])
    return run(users.astype(jnp.int32), items.astype(jnp.int32),
               t_i32, wsplit)


def kernel(users, items, embed_user, W):
    return _gmf_sc(users, items, embed_user, W)


# speedup vs baseline: 2.8691x; 2.8691x over previous
import functools

import jax
import jax.numpy as jnp
from jax import lax
from jax.experimental import pallas as pl
from jax.experimental.pallas import tpu as pltpu
from jax.experimental.pallas import tpu_sc as plsc

N_USERS = 1000000
D = 32
B = 16384
NC = 2
NS = 16
NW = NC * NS
BPW = B // NW
L = 16
NWAVE = BPW // L


def _perm(x, idx):
    dnums = lax.GatherDimensionNumbers(
        offset_dims=(), collapsed_slice_dims=(0,), start_index_map=(0,))
    return lax.gather(x, idx[:, None], dnums, slice_sizes=(1,),
                      mode=lax.GatherScatterMode.PROMISE_IN_BOUNDS)


@jax.jit
def _gmf_sc(users, items, embed_user, W):
    mesh = plsc.VectorSubcoreMesh(core_axis_name="c", subcore_axis_name="s")

    @functools.partial(
        pl.kernel,
        mesh=mesh,
        out_type=jax.ShapeDtypeStruct((B,), jnp.float32),
        compiler_params=pltpu.CompilerParams(use_tc_tiling_on_sc=True),
        scratch_types=[
            pltpu.VMEM((BPW,), jnp.int32),
            pltpu.VMEM((BPW,), jnp.int32),
            pltpu.VMEM((2, L, 16, D), jnp.bfloat16),
            pltpu.VMEM((2, L, 16, D), jnp.bfloat16),
            pltpu.VMEM((1, D), jnp.float32),
            pltpu.VMEM((BPW,), jnp.float32),
            pltpu.SemaphoreType.DMA,
            pltpu.SemaphoreType.DMA,
        ],
    )
    def run(users_hbm, items_hbm, table_hbm, w_hbm, out_hbm,
            uidx_v, iidx_v, ublk_v, iblk_v, w_v, out_v, sem0, sem1):
        wid = lax.axis_index("s") * NC + lax.axis_index("c")
        base = wid * BPW
        pltpu.sync_copy(users_hbm.at[pl.ds(base, BPW)], uidx_v)
        pltpu.sync_copy(items_hbm.at[pl.ds(base, BPW)], iidx_v)
        pltpu.sync_copy(w_hbm, w_v)

        def enqueue_wave(w, p, sem):
            uvec = uidx_v[pl.ds(w * L, L)]
            ivec = iidx_v[pl.ds(w * L, L)]
            for l in range(L):
                vu = uvec[l]
                offu = pl.multiple_of((vu >> 4) << 4, 16)
                pltpu.async_copy(
                    table_hbm.at[pl.ds(offu, 16), :], ublk_v.at[p, l], sem)
                vi = ivec[l]
                offi = pl.multiple_of((vi >> 4) << 4, 16)
                pltpu.async_copy(
                    table_hbm.at[pl.ds(offi, 16), :], iblk_v.at[p, l], sem)

        def drain_wave(p, sem):
            for l in range(L):
                pltpu.make_async_copy(
                    table_hbm.at[pl.ds(0, 16), :], ublk_v.at[p, l], sem).wait()
                pltpu.make_async_copy(
                    table_hbm.at[pl.ds(0, 16), :], iblk_v.at[p, l], sem).wait()

        w0 = w_v[0, pl.ds(0, L)]
        w1 = w_v[0, pl.ds(L, L)]
        lanes = lax.iota(jnp.int32, L)
        rot8 = lanes ^ 8
        rot4 = lanes ^ 4
        rot2 = lanes ^ 2
        rot1 = lanes ^ 1

        enqueue_wave(0, 0, sem0)

        def pair_step(h, _):
            we = h * 2
            wo = we + 1

            enqueue_wave(wo, 1, sem1)
            drain_wave(0, sem0)
            compute_wave(we, 0)

            @pl.when(h < NWAVE // 2 - 1)
            def _next_even():
                enqueue_wave(wo + 1, 0, sem0)

            drain_wave(1, sem1)
            compute_wave(wo, 1)
            return 0

        def compute_wave(w, p):
            uvec = uidx_v[pl.ds(w * L, L)]
            ivec = iidx_v[pl.ds(w * L, L)]
            acc = jnp.zeros((L,), jnp.float32)
            for l in range(L):
                ru = uvec[l] & 15
                rue = pl.multiple_of(ru & ~1, 2)
                pu = ru & 1
                ri = ivec[l] & 15
                rie = pl.multiple_of(ri & ~1, 2)
                pi = ri & 1
                up0 = ublk_v[p, l, pl.ds(rue, 2), pl.ds(0, L)].astype(jnp.float32)
                up1 = ublk_v[p, l, pl.ds(rue, 2), pl.ds(L, L)].astype(jnp.float32)
                ip0 = iblk_v[p, l, pl.ds(rie, 2), pl.ds(0, L)].astype(jnp.float32)
                ip1 = iblk_v[p, l, pl.ds(rie, 2), pl.ds(L, L)].astype(jnp.float32)
                u0 = jnp.where(pu == 0, up0[0], up0[1])
                u1 = jnp.where(pu == 0, up1[0], up1[1])
                i0 = jnp.where(pi == 0, ip0[0], ip0[1])
                i1 = jnp.where(pi == 0, ip1[0], ip1[1])
                s = u0 * i0 * w0 + u1 * i1 * w1
                s = s + _perm(s, rot8)
                s = s + _perm(s, rot4)
                s = s + _perm(s, rot2)
                s = s + _perm(s, rot1)
                acc = jnp.where(lanes == l, s, acc)
            out_v[pl.ds(w * L, L)] = acc

        lax.fori_loop(0, NWAVE // 2, pair_step, 0)

        pltpu.sync_copy(out_v, out_hbm.at[pl.ds(base, BPW)])

    return run(users.astype(jnp.int32), items.astype(jnp.int32),
               embed_user.astype(jnp.bfloat16), W)


def kernel(users, items, embed_user, W):
    return _gmf_sc(users, items, embed_user, W)


# speedup vs baseline: 2.9069x; 1.0132x over previous
import functools

import jax
import jax.numpy as jnp
from jax import lax
from jax.experimental import pallas as pl
from jax.experimental.pallas import tpu as pltpu
from jax.experimental.pallas import tpu_sc as plsc

N_USERS = 1000000
D = 32
B = 16384
NC = 2
NS = 16
NW = NC * NS
BPW = B // NW
L = 16
NWAVE = BPW // L


def _perm(x, idx):
    dnums = lax.GatherDimensionNumbers(
        offset_dims=(), collapsed_slice_dims=(0,), start_index_map=(0,))
    return lax.gather(x, idx[:, None], dnums, slice_sizes=(1,),
                      mode=lax.GatherScatterMode.PROMISE_IN_BOUNDS)


@jax.jit
def _gmf_sc(users, items, embed_user, W):
    mesh = plsc.VectorSubcoreMesh(core_axis_name="c", subcore_axis_name="s")

    @functools.partial(
        pl.kernel,
        mesh=mesh,
        out_type=jax.ShapeDtypeStruct((B,), jnp.float32),
        compiler_params=pltpu.CompilerParams(use_tc_tiling_on_sc=True),
        scratch_types=[
            pltpu.VMEM((BPW + L,), jnp.int32),
            pltpu.VMEM((BPW + L,), jnp.int32),
            pltpu.VMEM((4, 8, 16, D), jnp.bfloat16),
            pltpu.VMEM((4, 8, 16, D), jnp.bfloat16),
            pltpu.VMEM((1, D), jnp.float32),
            pltpu.VMEM((BPW,), jnp.float32),
            pltpu.SemaphoreType.DMA,
            pltpu.SemaphoreType.DMA,
            pltpu.SemaphoreType.DMA,
            pltpu.SemaphoreType.DMA,
        ],
    )
    def run(users_hbm, items_hbm, table_hbm, w_hbm, out_hbm,
            uidx_v, iidx_v, ublk_v, iblk_v, w_v, out_v,
            sem0, sem1, sem2, sem3):
        wid = lax.axis_index("s") * NC + lax.axis_index("c")
        base = wid * BPW
        pltpu.sync_copy(users_hbm.at[pl.ds(base, BPW)],
                        uidx_v.at[pl.ds(0, BPW)])
        pltpu.sync_copy(items_hbm.at[pl.ds(base, BPW)],
                        iidx_v.at[pl.ds(0, BPW)])
        pltpu.sync_copy(w_hbm, w_v)

        def enqueue_wave(w, p, sem):
            uvec = uidx_v[pl.ds(w * 8, L)]
            ivec = iidx_v[pl.ds(w * 8, L)]
            for l in range(8):
                vu = uvec[l]
                offu = pl.multiple_of((vu >> 4) << 4, 16)
                pltpu.async_copy(
                    table_hbm.at[pl.ds(offu, 16), :], ublk_v.at[p, l], sem)
                vi = ivec[l]
                offi = pl.multiple_of((vi >> 4) << 4, 16)
                pltpu.async_copy(
                    table_hbm.at[pl.ds(offi, 16), :], iblk_v.at[p, l], sem)

        def drain_wave(p, sem):
            for l in range(8):
                pltpu.make_async_copy(
                    table_hbm.at[pl.ds(0, 16), :], ublk_v.at[p, l], sem).wait()
                pltpu.make_async_copy(
                    table_hbm.at[pl.ds(0, 16), :], iblk_v.at[p, l], sem).wait()

        w0 = w_v[0, pl.ds(0, L)]
        w1 = w_v[0, pl.ds(L, L)]
        lanes = lax.iota(jnp.int32, L)
        rot8 = lanes ^ 8
        rot4 = lanes ^ 4
        rot2 = lanes ^ 2
        rot1 = lanes ^ 1

        sems = [sem0, sem1, sem2, sem3]
        NW8 = BPW // 8

        enqueue_wave(0, 0, sem0)
        enqueue_wave(1, 1, sem1)
        enqueue_wave(2, 2, sem2)

        def quad_step(h, _):
            wbase = h * 4
            acc = jnp.zeros((L,), jnp.float32)
            for k in range(4):
                w = wbase + k
                nxt = w + 3

                @pl.when(nxt < NW8)
                def _enqueue_ahead():
                    enqueue_wave(nxt, (k + 3) & 3, sems[(k + 3) & 3])

                drain_wave(k, sems[k])
                acc = compute_wave(w, k, k & 1, acc)
                if k & 1:
                    out_v[pl.ds((wbase + k - 1) * 8, L)] = acc
                    acc = jnp.zeros((L,), jnp.float32)
            return 0

        def compute_wave(w, p, half, acc):
            uvec = uidx_v[pl.ds(w * 8, L)]
            ivec = iidx_v[pl.ds(w * 8, L)]
            for l in range(8):
                ru = uvec[l] & 15
                rue = pl.multiple_of(ru & ~1, 2)
                pu = ru & 1
                ri = ivec[l] & 15
                rie = pl.multiple_of(ri & ~1, 2)
                pi = ri & 1
                up0 = ublk_v[p, l, pl.ds(rue, 2), pl.ds(0, L)].astype(jnp.float32)
                up1 = ublk_v[p, l, pl.ds(rue, 2), pl.ds(L, L)].astype(jnp.float32)
                ip0 = iblk_v[p, l, pl.ds(rie, 2), pl.ds(0, L)].astype(jnp.float32)
                ip1 = iblk_v[p, l, pl.ds(rie, 2), pl.ds(L, L)].astype(jnp.float32)
                u0 = jnp.where(pu == 0, up0[0], up0[1])
                u1 = jnp.where(pu == 0, up1[0], up1[1])
                i0 = jnp.where(pi == 0, ip0[0], ip0[1])
                i1 = jnp.where(pi == 0, ip1[0], ip1[1])
                s = u0 * i0 * w0 + u1 * i1 * w1
                s = s + _perm(s, rot8)
                s = s + _perm(s, rot4)
                s = s + _perm(s, rot2)
                s = s + _perm(s, rot1)
                acc = jnp.where(lanes == half * 8 + l, s, acc)
            return acc

        lax.fori_loop(0, BPW // 8 // 4, quad_step, 0)

        pltpu.sync_copy(out_v, out_hbm.at[pl.ds(base, BPW)])

    return run(users.astype(jnp.int32), items.astype(jnp.int32),
               embed_user.astype(jnp.bfloat16), W)


def kernel(users, items, embed_user, W):
    return _gmf_sc(users, items, embed_user, W)
